# baseline (speedup 1.0000x reference)
import functools

import jax
import jax.numpy as jnp
import numpy as np
from jax import lax
from jax.experimental import pallas as pl
from jax.experimental.pallas import tpu as pltpu
from jax.experimental.pallas import tpu_sc as plsc

NUM_TAGS = 44
B, L = 16384, 108
OUT_W = L + 1
LANES = 16
NC, NS = 2, 16
NW = NC * NS
ROWS_PER_W = B // NW
CHUNK = 128
N_CHUNKS = ROWS_PER_W // CHUNK
IN_CH = CHUNK * L
OUT_CH = CHUNK * OUT_W
SENT = IN_CH
TAB_PAD = 48


def _make_idx_map() -> np.ndarray:
    p = np.arange(OUT_CH)
    c = p % OUT_W
    return np.where(c == 0, SENT, p - p // OUT_W - 1).astype(np.int32)


_IDX_MAP = _make_idx_map()

_MESH = plsc.VectorSubcoreMesh(core_axis_name="c", subcore_axis_name="s")


@functools.partial(
    pl.kernel,
    out_type=jax.ShapeDtypeStruct((B * OUT_W,), jnp.int32),
    mesh=_MESH,
    compiler_params=pltpu.CompilerParams(needs_layout_passes=False),
    scratch_types=[
        pltpu.VMEM((TAB_PAD,), jnp.int32),
        pltpu.VMEM((OUT_CH,), jnp.int32),
        pltpu.VMEM((IN_CH + LANES,), jnp.int32),
        pltpu.VMEM((OUT_CH,), jnp.int32),
    ],
)
def _sc_lookup(tags_hbm, table_hbm, idxmap_hbm, out_hbm,
               tab_v, idx_v, tags_v, out_v):
    wid = lax.axis_index("s") * NC + lax.axis_index("c")
    pltpu.sync_copy(table_hbm, tab_v)
    pltpu.sync_copy(idxmap_hbm, idx_v)
    tags_v[pl.ds(SENT, LANES)] = jnp.full((LANES,), NUM_TAGS, jnp.int32)
    base = wid * ROWS_PER_W
    for ci in range(N_CHUNKS):
        r0 = base + ci * CHUNK
        pltpu.sync_copy(tags_hbm.at[pl.ds(r0 * L, IN_CH)],
                        tags_v.at[pl.ds(0, IN_CH)])

        def body(k, carry):
            idx16 = idx_v[pl.ds(k * LANES, LANES)]
            tag16 = plsc.load_gather(tags_v, [idx16])
            lab16 = plsc.load_gather(tab_v, [tag16])
            out_v[pl.ds(k * LANES, LANES)] = lab16
            return carry

        lax.fori_loop(0, OUT_CH // LANES, body, 0)
        pltpu.sync_copy(out_v, out_hbm.at[pl.ds(r0 * OUT_W, OUT_CH)])


def kernel(tags, label_table):
    tags_flat = tags.reshape(-1)
    table_pad = jnp.zeros((TAB_PAD,), jnp.int32).at[:NUM_TAGS].set(label_table)
    out_flat = _sc_lookup(tags_flat, table_pad, jnp.asarray(_IDX_MAP))
    return out_flat.reshape(B, OUT_W)


# speedup vs baseline: 1.0720x; 1.0720x over previous
import functools

import jax
import jax.numpy as jnp
import numpy as np
from jax import lax
from jax.experimental import pallas as pl
from jax.experimental.pallas import tpu as pltpu
from jax.experimental.pallas import tpu_sc as plsc

NUM_TAGS = 44
B, L = 16384, 108
OUT_W = L + 1
LANES = 16
NC, NS = 2, 16
NW = NC * NS
ROWS_PER_W = B // NW
CHUNK = 128
N_CHUNKS = ROWS_PER_W // CHUNK
IN_CH = CHUNK * L
OUT_CH = CHUNK * OUT_W
SENT = IN_CH
TAB_PAD = 48


def _make_idx_map() -> np.ndarray:
    p = np.arange(OUT_CH)
    c = p % OUT_W
    return np.where(c == 0, SENT, p - p // OUT_W - 1).astype(np.int32)


_IDX_MAP = _make_idx_map()

_MESH = plsc.VectorSubcoreMesh(core_axis_name="c", subcore_axis_name="s")


UNROLL = 8


@functools.partial(
    pl.kernel,
    out_type=jax.ShapeDtypeStruct((B * OUT_W,), jnp.int32),
    mesh=_MESH,
    compiler_params=pltpu.CompilerParams(needs_layout_passes=False),
    scratch_types=[
        pltpu.VMEM((TAB_PAD,), jnp.int32),
        pltpu.VMEM((OUT_CH,), jnp.int32),
        pltpu.VMEM((IN_CH + LANES,), jnp.int32),
        pltpu.VMEM((IN_CH + LANES,), jnp.int32),
        pltpu.VMEM((OUT_CH,), jnp.int32),
        pltpu.VMEM((OUT_CH,), jnp.int32),
        pltpu.SemaphoreType.DMA,
        pltpu.SemaphoreType.DMA,
        pltpu.SemaphoreType.DMA,
        pltpu.SemaphoreType.DMA,
    ],
)
def _sc_lookup(tags_hbm, table_hbm, idxmap_hbm, out_hbm,
               tab_v, idx_v, tags0, tags1, out0, out1,
               in_sem0, in_sem1, out_sem0, out_sem1):
    wid = lax.axis_index("s") * NC + lax.axis_index("c")
    base = wid * ROWS_PER_W
    tags_bufs, out_bufs = [tags0, tags1], [out0, out1]
    in_sems, out_sems = [in_sem0, in_sem1], [out_sem0, out_sem1]

    def start_in(ci):
        r0 = base + ci * CHUNK
        return pltpu.async_copy(tags_hbm.at[pl.ds(r0 * L, IN_CH)],
                                tags_bufs[ci % 2].at[pl.ds(0, IN_CH)],
                                in_sems[ci % 2])

    in_dma = [start_in(0), None]
    if N_CHUNKS > 1:
        in_dma[1] = start_in(1)
    pltpu.sync_copy(table_hbm, tab_v)
    pltpu.sync_copy(idxmap_hbm, idx_v)
    tags0[pl.ds(SENT, LANES)] = jnp.full((LANES,), NUM_TAGS, jnp.int32)
    tags1[pl.ds(SENT, LANES)] = jnp.full((LANES,), NUM_TAGS, jnp.int32)

    out_dma = [None, None]
    for ci in range(N_CHUNKS):
        b = ci % 2
        in_dma[b].wait()
        if out_dma[b] is not None:
            out_dma[b].wait()
        tbuf, obuf = tags_bufs[b], out_bufs[b]

        def body(k, carry):
            vbase = k * (LANES * UNROLL)
            for u in range(UNROLL):
                off = vbase + u * LANES
                idx16 = idx_v[pl.ds(off, LANES)]
                tag16 = plsc.load_gather(tbuf, [idx16])
                lab16 = plsc.load_gather(tab_v, [tag16])
                obuf[pl.ds(off, LANES)] = lab16
            return carry

        lax.fori_loop(0, OUT_CH // (LANES * UNROLL), body, 0)
        if ci + 2 < N_CHUNKS:
            in_dma[b] = start_in(ci + 2)
        r0 = base + ci * CHUNK
        out_dma[b] = pltpu.async_copy(
            obuf, out_hbm.at[pl.ds(r0 * OUT_W, OUT_CH)], out_sems[b])
    for b in range(min(2, N_CHUNKS)):
        out_dma[b].wait()


def kernel(tags, label_table):
    tags_flat = tags.reshape(-1)
    table_pad = jnp.zeros((TAB_PAD,), jnp.int32).at[:NUM_TAGS].set(label_table)
    out_flat = _sc_lookup(tags_flat, table_pad, jnp.asarray(_IDX_MAP))
    return out_flat.reshape(B, OUT_W)


# speedup vs baseline: 1.4113x; 1.3165x over previous
import functools

import jax
import jax.numpy as jnp
import numpy as np
from jax import lax
from jax.experimental import pallas as pl
from jax.experimental.pallas import tpu as pltpu
from jax.experimental.pallas import tpu_sc as plsc

NUM_TAGS = 44
B, L = 16384, 108
OUT_W = L + 1
LANES = 16
NC, NS = 2, 16
NW = NC * NS
ROWS_PER_W = B // NW
CHUNK = 128
N_CHUNKS = ROWS_PER_W // CHUNK
IN_CH = CHUNK * L
OUT_CH = CHUNK * OUT_W
SENT = IN_CH
TAB_PAD = 48


def _make_idx_map() -> np.ndarray:
    p = np.arange(OUT_CH)
    c = p % OUT_W
    return np.where(c == 0, SENT, p - p // OUT_W - 1).astype(np.int32)


_IDX_MAP = _make_idx_map()

_MESH = plsc.VectorSubcoreMesh(core_axis_name="c", subcore_axis_name="s")


UNROLL = 8


@functools.partial(
    pl.kernel,
    out_type=jax.ShapeDtypeStruct((B * OUT_W,), jnp.int32),
    mesh=_MESH,
    compiler_params=pltpu.CompilerParams(needs_layout_passes=False),
    scratch_types=[
        pltpu.VMEM((TAB_PAD,), jnp.int32),
        pltpu.VMEM((OUT_CH,), jnp.int32),
        pltpu.VMEM((IN_CH + LANES,), jnp.int32),
        pltpu.VMEM((IN_CH + LANES,), jnp.int32),
        pltpu.VMEM((OUT_CH,), jnp.int32),
        pltpu.VMEM((OUT_CH,), jnp.int32),
        pltpu.SemaphoreType.DMA,
        pltpu.SemaphoreType.DMA,
        pltpu.SemaphoreType.DMA,
        pltpu.SemaphoreType.DMA,
    ],
)
def _sc_lookup(tags_hbm, table_hbm, idxmap_hbm, out_hbm,
               tab_v, idx_v, tags0, tags1, out0, out1,
               in_sem0, in_sem1, out_sem0, out_sem1):
    wid = lax.axis_index("s") * NC + lax.axis_index("c")
    base = wid * ROWS_PER_W
    tags_bufs, out_bufs = [tags0, tags1], [out0, out1]
    in_sems, out_sems = [in_sem0, in_sem1], [out_sem0, out_sem1]

    def start_in(ci):
        r0 = base + ci * CHUNK
        return pltpu.async_copy(tags_hbm.at[pl.ds(r0 * L, IN_CH)],
                                tags_bufs[ci % 2].at[pl.ds(0, IN_CH)],
                                in_sems[ci % 2])

    in_dma = [start_in(0), None]
    if N_CHUNKS > 1:
        in_dma[1] = start_in(1)
    pltpu.sync_copy(table_hbm, tab_v)
    pltpu.sync_copy(idxmap_hbm, idx_v)
    tags0[pl.ds(SENT, LANES)] = jnp.full((LANES,), NUM_TAGS, jnp.int32)
    tags1[pl.ds(SENT, LANES)] = jnp.full((LANES,), NUM_TAGS, jnp.int32)

    out_dma = [None, None]
    for ci in range(N_CHUNKS):
        b = ci % 2
        in_dma[b].wait()
        if out_dma[b] is not None:
            out_dma[b].wait()
        tbuf, obuf = tags_bufs[b], out_bufs[b]

        @plsc.parallel_loop(0, OUT_CH // LANES, 1, unroll=UNROLL)
        def _gather_body(k):
            off = k * LANES
            idx16 = idx_v[pl.ds(off, LANES)]
            tag16 = plsc.load_gather(tbuf, [idx16])
            lab16 = plsc.load_gather(tab_v, [tag16])
            obuf[pl.ds(off, LANES)] = lab16
        if ci + 2 < N_CHUNKS:
            in_dma[b] = start_in(ci + 2)
        r0 = base + ci * CHUNK
        out_dma[b] = pltpu.async_copy(
            obuf, out_hbm.at[pl.ds(r0 * OUT_W, OUT_CH)], out_sems[b])
    for b in range(min(2, N_CHUNKS)):
        out_dma[b].wait()


def kernel(tags, label_table):
    tags_flat = tags.reshape(-1)
    table_pad = jnp.zeros((TAB_PAD,), jnp.int32).at[:NUM_TAGS].set(label_table)
    out_flat = _sc_lookup(tags_flat, table_pad, jnp.asarray(_IDX_MAP))
    return out_flat.reshape(B, OUT_W)


# speedup vs baseline: 2.1524x; 1.5251x over previous
import functools

import jax
import jax.numpy as jnp
import numpy as np
from jax import lax
from jax.experimental import pallas as pl
from jax.experimental.pallas import tpu as pltpu
from jax.experimental.pallas import tpu_sc as plsc

NUM_TAGS = 44
B, L = 16384, 108
OUT_W = L + 1
LANES = 16
NC, NS = 2, 16
NW = NC * NS
ROWS_PER_W = B // NW
CHUNK = 128
N_CHUNKS = ROWS_PER_W // CHUNK
OUT_CH = CHUNK * OUT_W
SENT_ROW = CHUNK
TAB_PAD = 48
UNROLL = 8


def _make_maps() -> np.ndarray:
    p = np.arange(OUT_CH)
    return np.stack([p // OUT_W, p % OUT_W]).astype(np.int32)


_MAPS = _make_maps()

_MESH = plsc.VectorSubcoreMesh(core_axis_name="c", subcore_axis_name="s")


@functools.partial(
    pl.kernel,
    out_type=jax.ShapeDtypeStruct((B, OUT_W), jnp.int32),
    mesh=_MESH,
    compiler_params=pltpu.CompilerParams(needs_layout_passes=False),
    scratch_types=[
        pltpu.VMEM((TAB_PAD,), jnp.int32),
        pltpu.VMEM((OUT_CH,), jnp.int32),
        pltpu.VMEM((OUT_CH,), jnp.int32),
        pltpu.VMEM((CHUNK + 1, L), jnp.int32),
        pltpu.VMEM((CHUNK + 1, L), jnp.int32),
        pltpu.VMEM((CHUNK, OUT_W), jnp.int32),
        pltpu.VMEM((CHUNK, OUT_W), jnp.int32),
        pltpu.SemaphoreType.DMA,
        pltpu.SemaphoreType.DMA,
        pltpu.SemaphoreType.DMA,
        pltpu.SemaphoreType.DMA,
    ],
)
def _sc_lookup(tags_hbm, table_hbm, maps_hbm, out_hbm,
               tab_v, rmap_v, cmap_v, tags0, tags1, out0, out1,
               in_sem0, in_sem1, out_sem0, out_sem1):
    wid = lax.axis_index("s") * NC + lax.axis_index("c")
    base = wid * ROWS_PER_W
    tags_bufs, out_bufs = [tags0, tags1], [out0, out1]
    in_sems, out_sems = [in_sem0, in_sem1], [out_sem0, out_sem1]

    def start_in(ci):
        r0 = base + ci * CHUNK
        return pltpu.async_copy(tags_hbm.at[pl.ds(r0, CHUNK)],
                                tags_bufs[ci % 2].at[pl.ds(0, CHUNK)],
                                in_sems[ci % 2])

    in_dma = [start_in(0), None]
    if N_CHUNKS > 1:
        in_dma[1] = start_in(1)
    pltpu.sync_copy(table_hbm, tab_v)
    pltpu.sync_copy(maps_hbm.at[0], rmap_v)
    pltpu.sync_copy(maps_hbm.at[1], cmap_v)
    sent_rows = jnp.full((LANES,), SENT_ROW, jnp.int32)
    sent_cols = lax.iota(jnp.int32, LANES)
    sent_vals = jnp.full((LANES,), NUM_TAGS, jnp.int32)
    plsc.store_scatter(tags0, [sent_rows, sent_cols], sent_vals)
    plsc.store_scatter(tags1, [sent_rows, sent_cols], sent_vals)

    out_dma = [None, None]
    for ci in range(N_CHUNKS):
        b = ci % 2
        in_dma[b].wait()
        if out_dma[b] is not None:
            out_dma[b].wait()
        tbuf, obuf = tags_bufs[b], out_bufs[b]

        @plsc.parallel_loop(0, OUT_CH // LANES, 1, unroll=UNROLL)
        def _gather_body(k):
            off = k * LANES
            row16 = rmap_v[pl.ds(off, LANES)]
            ocol16 = cmap_v[pl.ds(off, LANES)]
            is_pad = ocol16 == 0
            irow16 = jnp.where(is_pad, SENT_ROW, row16)
            icol16 = jnp.where(is_pad, 0, ocol16 - 1)
            tag16 = plsc.load_gather(tbuf, [irow16, icol16])
            lab16 = plsc.load_gather(tab_v, [tag16])
            plsc.store_scatter(obuf, [row16, ocol16], lab16)

        if ci + 2 < N_CHUNKS:
            in_dma[b] = start_in(ci + 2)
        r0 = base + ci * CHUNK
        out_dma[b] = pltpu.async_copy(
            obuf, out_hbm.at[pl.ds(r0, CHUNK)], out_sems[b])
    for b in range(min(2, N_CHUNKS)):
        out_dma[b].wait()


def kernel(tags, label_table):
    table_pad = jnp.zeros((TAB_PAD,), jnp.int32).at[:NUM_TAGS].set(label_table)
    return _sc_lookup(tags, table_pad, jnp.asarray(_MAPS))


# speedup vs baseline: 2.5406x; 1.1804x over previous
import functools

import jax
import jax.numpy as jnp
import numpy as np
from jax import lax
from jax.experimental import pallas as pl
from jax.experimental.pallas import tpu as pltpu
from jax.experimental.pallas import tpu_sc as plsc

NUM_TAGS = 44
B, L = 16384, 108
OUT_W = L + 1
LANES = 16
NC, NS = 2, 16
NW = NC * NS
ROWS_PER_W = B // NW
CHUNK = 128
N_CHUNKS = ROWS_PER_W // CHUNK
OUT_CH = CHUNK * OUT_W
SENT_ROW = CHUNK
TAB_PAD = 48
UNROLL = 2


_ROW_OFFS = (0, 16, 32, 48, 64, 80, L - LANES)

_MESH = plsc.VectorSubcoreMesh(core_axis_name="c", subcore_axis_name="s")


@functools.partial(
    pl.kernel,
    out_type=jax.ShapeDtypeStruct((B, OUT_W), jnp.int32),
    mesh=_MESH,
    compiler_params=pltpu.CompilerParams(needs_layout_passes=False),
    scratch_types=[
        pltpu.VMEM((TAB_PAD,), jnp.int32),
        pltpu.VMEM((CHUNK, L), jnp.int32),
        pltpu.VMEM((CHUNK, L), jnp.int32),
        pltpu.VMEM((CHUNK, OUT_W), jnp.int32),
        pltpu.VMEM((CHUNK, OUT_W), jnp.int32),
        pltpu.SemaphoreType.DMA,
        pltpu.SemaphoreType.DMA,
        pltpu.SemaphoreType.DMA,
        pltpu.SemaphoreType.DMA,
    ],
)
def _sc_lookup(tags_hbm, table_hbm, out_hbm,
               tab_v, tags0, tags1, out0, out1,
               in_sem0, in_sem1, out_sem0, out_sem1):
    wid = lax.axis_index("s") * NC + lax.axis_index("c")
    base = wid * ROWS_PER_W
    tags_bufs, out_bufs = [tags0, tags1], [out0, out1]
    in_sems, out_sems = [in_sem0, in_sem1], [out_sem0, out_sem1]

    def start_in(ci):
        r0 = base + ci * CHUNK
        return pltpu.async_copy(tags_hbm.at[pl.ds(r0, CHUNK)],
                                tags_bufs[ci % 2],
                                in_sems[ci % 2])

    in_dma = [start_in(0), None]
    if N_CHUNKS > 1:
        in_dma[1] = start_in(1)
    pltpu.sync_copy(table_hbm, tab_v)
    zero_vals = jnp.zeros((LANES,), jnp.int32)
    zero_cols = jnp.zeros((LANES,), jnp.int32)
    for i in range(CHUNK // LANES):
        rows16 = lax.iota(jnp.int32, LANES) + i * LANES
        plsc.store_scatter(out0, [rows16, zero_cols], zero_vals)
        plsc.store_scatter(out1, [rows16, zero_cols], zero_vals)

    out_dma = [None, None]
    for ci in range(N_CHUNKS):
        b = ci % 2
        in_dma[b].wait()
        if out_dma[b] is not None:
            out_dma[b].wait()
        tbuf, obuf = tags_bufs[b], out_bufs[b]

        @plsc.parallel_loop(0, CHUNK, 1, unroll=UNROLL)
        def _gather_row(r):
            trow, orow = tbuf.at[r], obuf.at[r]
            for off in _ROW_OFFS:
                tag16 = trow[pl.ds(off, LANES)]
                lab16 = plsc.load_gather(tab_v, [tag16])
                orow[pl.ds(off + 1, LANES)] = lab16

        if ci + 2 < N_CHUNKS:
            in_dma[b] = start_in(ci + 2)
        r0 = base + ci * CHUNK
        out_dma[b] = pltpu.async_copy(
            obuf, out_hbm.at[pl.ds(r0, CHUNK)], out_sems[b])
    for b in range(min(2, N_CHUNKS)):
        out_dma[b].wait()


def kernel(tags, label_table):
    table_pad = jnp.zeros((TAB_PAD,), jnp.int32).at[:NUM_TAGS].set(label_table)
    return _sc_lookup(tags, table_pad)
